# baseline (speedup 1.0000x reference)
import jax
import jax.numpy as jnp
from jax.experimental import pallas as pl
from jax.experimental.pallas import tpu as pltpu

_NBUF = 3


def _dwt_l1_body(p_hbm, t_hbm, out_ref, pbuf, tbuf, psem, tsem):
    nbuf, bb, cc, h, w = pbuf.shape
    n_local = pl.num_programs(1)
    ci = pl.program_id(0)
    i = pl.program_id(1)
    base = ci * n_local

    def start(k_local, slot):
        blk = (base + k_local) * bb
        pltpu.make_async_copy(
            p_hbm.at[pl.ds(blk, bb)], pbuf.at[slot], psem.at[slot]).start()
        pltpu.make_async_copy(
            t_hbm.at[pl.ds(blk, bb)], tbuf.at[slot], tsem.at[slot]).start()

    @pl.when(i == 0)
    def _prologue():
        for k in range(_NBUF - 1):
            start(k, k)

    nxt = i + _NBUF - 1

    @pl.when(nxt < n_local)
    def _lookahead():
        start(nxt, jax.lax.rem(nxt, _NBUF))

    slot = jax.lax.rem(i, _NBUF)
    pltpu.make_async_copy(
        pbuf.at[slot], pbuf.at[slot], psem.at[slot]).wait()
    pltpu.make_async_copy(
        tbuf.at[slot], tbuf.at[slot], tsem.at[slot]).wait()

    g = 128
    L = 128
    acc = jnp.zeros((g // 8, 8, L), jnp.float32)
    for b in range(bb):
        for c in range(cc):
            for r0 in range(0, h, g):
                for c0 in range(0, w, L):
                    e = (pbuf[slot, b, c, r0:r0 + g, c0:c0 + L]
                         - tbuf[slot, b, c, r0:r0 + g, c0:c0 + L])
                    e = e.reshape(g // 8, 8, L)
                    e_dn = pltpu.roll(e, 1, 1)
                    av = jnp.abs(e + e_dn)
                    aw = jnp.abs(e - e_dn)
                    avr = pltpu.roll(av, L - 1, 2)
                    awr = pltpu.roll(aw, L - 1, 2)
                    acc = acc + jnp.maximum(av, avr) + jnp.maximum(aw, awr)
    row = jax.lax.broadcasted_iota(jnp.int32, (1, 8, L), 1)
    acc = jnp.where((row & 1) == 1, acc, 0.0)
    colsum = jnp.sum(acc.reshape(g, L), axis=0, keepdims=True)
    lane = jax.lax.broadcasted_iota(jnp.int32, colsum.shape, 1)
    masked = jnp.where((lane & 1) == 0, colsum, 0.0)
    out_ref[...] = jnp.sum(masked, axis=1, keepdims=True)[None, None]


def kernel(pred, target):
    B, C, H, W = pred.shape

    BB = 2
    N = B // BB
    steps = N // 2

    any_spec = pl.BlockSpec(memory_space=pl.ANY)

    partials = pl.pallas_call(
        _dwt_l1_body,
        grid=(2, steps),
        in_specs=[any_spec, any_spec],
        out_specs=pl.BlockSpec(
            (1, 1, 1, 1), lambda ci, i: (ci * steps + i, 0, 0, 0)),
        out_shape=jax.ShapeDtypeStruct((N, 1, 1, 1), jnp.float32),
        scratch_shapes=[
            pltpu.VMEM((_NBUF, BB, C, H, W), jnp.float32),
            pltpu.VMEM((_NBUF, BB, C, H, W), jnp.float32),
            pltpu.SemaphoreType.DMA((_NBUF,)),
            pltpu.SemaphoreType.DMA((_NBUF,)),
        ],
        compiler_params=pltpu.CompilerParams(
            dimension_semantics=("parallel", "arbitrary"),
            vmem_limit_bytes=62 * 1024 * 1024,
        ),
    )(pred, target)

    n = B * C * (H // 2) * (W // 2)
    return jnp.sum(partials) * (1.0 / n)
